# baseline (speedup 1.0000x reference)
import functools

import jax
import jax.numpy as jnp
from jax import lax
from jax.experimental import pallas as pl
from jax.experimental.pallas import tpu as pltpu
from jax.experimental.pallas import tpu_sc as plsc

NUM_EMB = 1000000
DIM = 32
NROW = 16384
NCOL = 26

NC = 2
NS = 16
NW = NC * NS

NTILE = NUM_EMB // 128
TPW = NTILE // NW
NEXTRA = NTILE - TPW * NW
TAIL0 = NTILE * 128
NTAIL = NUM_EMB - TAIL0


def _linearize_body(wt_hbm, tail_hbm, wf_hbm,
                    v0, v1, v2, v3, t0, t1, t2, t3,
                    gs0, gs1, gs2, gs3, ws0, ws1, ws2, ws3):
    wid = lax.axis_index("s") * NC + lax.axis_index("c")
    it0 = wid * TPW
    vb = (v0, v1, v2, v3)
    tb = (t0, t1, t2, t3)
    gs = (gs0, gs1, gs2, gs3)
    ws = (ws0, ws1, ws2, ws3)
    iota = lax.iota(jnp.int32, 16)
    rows_lo = iota
    rows_hi = iota + 16
    zeros = jnp.zeros((16,), jnp.int32)

    def start_gather(slot, it):
        return pltpu.async_copy(
            wt_hbm.at[:, pl.ds(it * 128, 128)], vb[slot], gs[slot])

    def start_write(slot, it):
        return pltpu.async_copy(
            tb[slot], wf_hbm.at[pl.ds(it * 4096, 4096)], ws[slot])

    def drain_write(slot):
        pltpu.make_async_copy(
            wf_hbm.at[pl.ds(0, 4096)], tb[slot], ws[slot]).wait()

    def drain_gather(slot):
        pltpu.make_async_copy(
            wt_hbm.at[:, pl.ds(0, 128)], vb[slot], gs[slot]).wait()

    def transpose(slot):
        src = vb[slot]
        dst = tb[slot]

        def _tr(base, carry):
            for u in range(8):
                i = base * 8 + u
                col = zeros + i
                dst[pl.ds(i * 32, 16)] = plsc.load_gather(
                    src, [rows_lo, col])
                dst[pl.ds(i * 32 + 16, 16)] = plsc.load_gather(
                    src, [rows_hi, col])
            return carry
        lax.fori_loop(0, 16, _tr, 0)

    gh = [start_gather(b, it0 + b) for b in range(4)]
    for b in range(4):
        gh[b].wait()
        transpose(b)
        start_write(b, it0 + b)
        start_gather(b, it0 + 4 + b)

    def step(base, carry):
        for b in range(4):
            it = base * 4 + b
            drain_write(b)
            drain_gather(b)
            transpose(b)
            start_write(b, it0 + it)
            start_gather(b, it0 + it + 4)
        return carry
    lax.fori_loop(1, 60, step, 0)

    for b in range(4):
        drain_write(b)
        drain_gather(b)
        transpose(b)
        start_write(b, it0 + 240 + b)
    for b in range(4):
        drain_write(b)

    @pl.when(wid < NEXTRA)
    def _extras():
        it = TPW * NW + wid
        pltpu.sync_copy(wt_hbm.at[:, pl.ds(it * 128, 128)], vb[0])
        transpose(0)
        pltpu.sync_copy(tb[0], wf_hbm.at[pl.ds(it * 4096, 4096)])

    @pl.when(wid == NEXTRA)
    def _tail():
        pltpu.sync_copy(tail_hbm, t0.at[pl.ds(0, NTAIL * DIM)])
        pltpu.sync_copy(t0.at[pl.ds(0, NTAIL * DIM)],
                        wf_hbm.at[pl.ds(TAIL0 * DIM, NTAIL * DIM)])


def _linearize(wt, tail):
    mesh = plsc.VectorSubcoreMesh(core_axis_name="c", subcore_axis_name="s")
    k = pl.kernel(
        _linearize_body,
        mesh=mesh,
        compiler_params=pltpu.CompilerParams(
            use_tc_tiling_on_sc=True, needs_layout_passes=False),
        out_type=jax.ShapeDtypeStruct((NUM_EMB * DIM,), jnp.float32),
        scratch_types=(
            [pltpu.VMEM((32, 128), jnp.float32) for _ in range(4)]
            + [pltpu.VMEM((4096,), jnp.float32) for _ in range(4)]
            + [pltpu.SemaphoreType.DMA for _ in range(8)]
        ),
    )
    return k(wt, tail)


RPT = NROW // NW
BTPT = RPT // 128


def _gather_body(idx_hbm, w2d_hbm, out_hbm,
                 idx_v, idxT, g0, g1, tb0, tb1,
                 gsem0, gsem1, ssem0, ssem1):
    wid = lax.axis_index("s") * NC + lax.axis_index("c")
    b0 = wid * RPT
    iota = lax.iota(jnp.int32, 16)
    zeros = jnp.zeros((16,), jnp.int32)
    gb = (g0, g1)
    tbufs = (tb0, tb1)
    gsems = (gsem0, gsem1)
    ssems = (ssem0, ssem1)

    pltpu.sync_copy(idx_hbm.at[pl.ds(b0, RPT)], idx_v)

    def _tr_idx(base, carry):
        for u in range(8):
            r = base * 8 + u
            c = r >> 5
            g = r & 31
            v = plsc.load_gather(idx_v, [g * 16 + iota, zeros + c])
            idxT[pl.ds(c * 512 + g * 16, 16)] = v
        return carry
    lax.fori_loop(0, 26 * 4, _tr_idx, 0)

    def start_gather(p, c):
        return pltpu.async_copy(
            w2d_hbm.at[idxT.at[pl.ds(c * 512, 512)]], gb[p], gsems[p])

    def drain_scatters(p):
        pltpu.make_async_copy(
            w2d_hbm.at[pl.ds(0, RPT)], gb[p], ssems[p]).wait()

    rows8 = tuple(iota + (blg * 16) for blg in range(8))

    def transpose_rows(p):
        src = gb[p]
        dst = tbufs[p]

        def _tr(t, carry):
            btl = t >> 5
            j = t & 31
            col = zeros + j
            rbase = btl * 128
            dbase = (btl * 32 + j) * 128
            for blg in range(8):
                v = plsc.load_gather(src, [rbase + rows8[blg], col])
                dst[pl.ds(dbase + blg * 16, 16)] = v
            return carry
        lax.fori_loop(0, 128, _tr, 0)

    def scatter_out(p, c):
        src = tbufs[p]

        def _sc(r, carry):
            btl = r >> 5
            jo = (r >> 3) & 3
            jr = r & 7
            pltpu.async_copy(
                src.at[pl.ds(r * 128, 128)],
                out_hbm.at[c, jo, wid * BTPT + btl, jr], ssems[p])
            return carry
        lax.fori_loop(0, 128, _sc, 0)

    gh = [None, None]
    gh[0] = start_gather(0, 0)
    for c in range(NCOL):
        p = c & 1
        np_ = p ^ 1
        if c + 1 < NCOL:
            if c >= 1:
                drain_scatters(np_)
            gh[np_] = start_gather(np_, c + 1)
        gh[p].wait()
        transpose_rows(p)
        scatter_out(p, c)
    drain_scatters(0)
    drain_scatters(1)


def _gather(idx, w2d):
    mesh = plsc.VectorSubcoreMesh(core_axis_name="c", subcore_axis_name="s")
    k = pl.kernel(
        _gather_body,
        mesh=mesh,
        compiler_params=pltpu.CompilerParams(
            use_tc_tiling_on_sc=False, needs_layout_passes=False),
        out_type=jax.ShapeDtypeStruct((NCOL, 4, 128, 8, 128), jnp.float32),
        scratch_types=[
            pltpu.VMEM((RPT, NCOL), jnp.int32),
            pltpu.VMEM((NCOL * RPT,), jnp.int32),
            pltpu.VMEM((RPT, DIM), jnp.float32),
            pltpu.VMEM((RPT, DIM), jnp.float32),
            pltpu.VMEM((RPT * DIM,), jnp.float32),
            pltpu.VMEM((RPT * DIM,), jnp.float32),
            pltpu.SemaphoreType.DMA,
            pltpu.SemaphoreType.DMA,
            pltpu.SemaphoreType.DMA,
            pltpu.SemaphoreType.DMA,
        ],
    )
    return k(idx, w2d)


def kernel(inputs, weight):
    idx = inputs.astype(jnp.int32)
    tail = lax.slice(weight, (TAIL0, 0), (NUM_EMB, DIM)).reshape(-1)
    w1d = _linearize(weight.T, tail)
    w2d = w1d.reshape(NUM_EMB, DIM)
    out5 = _gather(idx, w2d)
    return jnp.transpose(out5, (2, 4, 0, 1, 3)).reshape(NROW, NCOL, DIM)


# speedup vs baseline: 5.6440x; 5.6440x over previous
import functools

import jax
import jax.numpy as jnp
from jax import lax
from jax.experimental import pallas as pl
from jax.experimental.pallas import tpu as pltpu
from jax.experimental.pallas import tpu_sc as plsc

NUM_EMB = 1000000
DIM = 32
NROW = 16384
NCOL = 26

NC = 2
NS = 16
NW = NC * NS

NTILE = NUM_EMB // 128
TPW = NTILE // NW
NEXTRA = NTILE - TPW * NW
TAIL0 = NTILE * 128
NTAIL = NUM_EMB - TAIL0


def _linearize_body(wt_hbm, tail_hbm, wf_hbm,
                    v0, v1, v2, v3, t0, t1, t2, t3,
                    gs0, gs1, gs2, gs3, ws0, ws1, ws2, ws3):
    wid = lax.axis_index("s") * NC + lax.axis_index("c")
    it0 = wid * TPW
    vb = (v0, v1, v2, v3)
    tb = (t0, t1, t2, t3)
    gs = (gs0, gs1, gs2, gs3)
    ws = (ws0, ws1, ws2, ws3)
    iota = lax.iota(jnp.int32, 16)
    rows_lo = iota
    rows_hi = iota + 16
    zeros = jnp.zeros((16,), jnp.int32)

    def start_gather(slot, it):
        return pltpu.async_copy(
            wt_hbm.at[:, pl.ds(it * 128, 128)], vb[slot], gs[slot])

    def start_write(slot, it):
        return pltpu.async_copy(
            tb[slot], wf_hbm.at[pl.ds(it * 4096, 4096)], ws[slot])

    def drain_write(slot):
        pltpu.make_async_copy(
            wf_hbm.at[pl.ds(0, 4096)], tb[slot], ws[slot]).wait()

    def drain_gather(slot):
        pltpu.make_async_copy(
            wt_hbm.at[:, pl.ds(0, 128)], vb[slot], gs[slot]).wait()

    rot = tuple((iota + k) & 15 for k in range(16))

    def transpose(slot):
        src = vb[slot]
        dst = tb[slot]

        def _tr(bi, carry):
            i0 = (bi >> 1) * 16
            j0 = (bi & 1) * 16
            rowv = j0 + iota
            vals = [plsc.load_gather(src, [rowv, i0 + rot[k]])
                    for k in range(16)]
            dbase = i0 * 32 + j0
            for k in range(16):
                plsc.store_scatter(
                    dst, [rot[k] * 32 + (dbase + iota)], vals[k])
            return carry
        lax.fori_loop(0, 16, _tr, 0)

    gh = [start_gather(b, it0 + b) for b in range(4)]
    for b in range(4):
        gh[b].wait()
        transpose(b)
        start_write(b, it0 + b)
        start_gather(b, it0 + 4 + b)

    def step(base, carry):
        for b in range(4):
            it = base * 4 + b
            drain_write(b)
            drain_gather(b)
            transpose(b)
            start_write(b, it0 + it)
            start_gather(b, it0 + it + 4)
        return carry
    lax.fori_loop(1, 60, step, 0)

    for b in range(4):
        drain_write(b)
        drain_gather(b)
        transpose(b)
        start_write(b, it0 + 240 + b)
    for b in range(4):
        drain_write(b)

    @pl.when(wid < NEXTRA)
    def _extras():
        it = TPW * NW + wid
        pltpu.sync_copy(wt_hbm.at[:, pl.ds(it * 128, 128)], vb[0])
        transpose(0)
        pltpu.sync_copy(tb[0], wf_hbm.at[pl.ds(it * 4096, 4096)])

    @pl.when(wid == NEXTRA)
    def _tail():
        pltpu.sync_copy(tail_hbm, t0.at[pl.ds(0, NTAIL * DIM)])
        pltpu.sync_copy(t0.at[pl.ds(0, NTAIL * DIM)],
                        wf_hbm.at[pl.ds(TAIL0 * DIM, NTAIL * DIM)])


def _linearize(wt, tail):
    mesh = plsc.VectorSubcoreMesh(core_axis_name="c", subcore_axis_name="s")
    k = pl.kernel(
        _linearize_body,
        mesh=mesh,
        compiler_params=pltpu.CompilerParams(
            use_tc_tiling_on_sc=True, needs_layout_passes=False),
        out_type=jax.ShapeDtypeStruct((NUM_EMB * DIM,), jnp.float32),
        scratch_types=(
            [pltpu.VMEM((32, 128), jnp.float32) for _ in range(4)]
            + [pltpu.VMEM((4096,), jnp.float32) for _ in range(4)]
            + [pltpu.SemaphoreType.DMA for _ in range(8)]
        ),
    )
    return k(wt, tail)


RPT = NROW // NW
BTPT = RPT // 128


def _gather_body(idx_hbm, w2d_hbm, out_hbm,
                 idx_v, idxT, g0, g1, tb0, tb1,
                 gsem0, gsem1, ssem0, ssem1):
    wid = lax.axis_index("s") * NC + lax.axis_index("c")
    b0 = wid * RPT
    iota = lax.iota(jnp.int32, 16)
    zeros = jnp.zeros((16,), jnp.int32)
    gb = (g0, g1)
    tbufs = (tb0, tb1)
    gsems = (gsem0, gsem1)
    ssems = (ssem0, ssem1)

    pltpu.sync_copy(idx_hbm.at[pl.ds(b0, RPT)], idx_v)

    def _tr_idx(base, carry):
        for u in range(8):
            r = base * 8 + u
            c = r >> 5
            g = r & 31
            v = plsc.load_gather(idx_v, [g * 16 + iota, zeros + c])
            idxT[pl.ds(c * 512 + g * 16, 16)] = v
        return carry
    lax.fori_loop(0, 26 * 4, _tr_idx, 0)

    def start_gather(p, c):
        return pltpu.async_copy(
            w2d_hbm.at[idxT.at[pl.ds(c * 512, 512)]], gb[p], gsems[p])

    def drain_scatters(p):
        pltpu.make_async_copy(
            w2d_hbm.at[pl.ds(0, RPT)], gb[p], ssems[p]).wait()

    rot = tuple((iota + k) & 15 for k in range(16))
    l128 = iota * 128

    def transpose_rows(p):
        src = gb[p]
        dst = tbufs[p]

        def _tr(bi, carry):
            i0 = (bi >> 1) * 16
            j0 = (bi & 1) * 16
            btl = i0 >> 7
            colv = j0 + iota
            vals = [plsc.load_gather(src, [i0 + rot[k], colv])
                    for k in range(16)]
            dl = (btl * 4096 + j0 * 128 + (i0 & 127)) + l128
            for k in range(16):
                plsc.store_scatter(dst, [dl + rot[k]], vals[k])
            return carry
        lax.fori_loop(0, 64, _tr, 0)

    def scatter_out(p, c):
        src = tbufs[p]

        def _sc(r, carry):
            btl = r >> 5
            jo = (r >> 3) & 3
            jr = r & 7
            pltpu.async_copy(
                src.at[pl.ds(r * 128, 128)],
                out_hbm.at[c, jo, wid * BTPT + btl, jr], ssems[p])
            return carry
        lax.fori_loop(0, 128, _sc, 0)

    gh = [None, None]
    gh[0] = start_gather(0, 0)
    for c in range(NCOL):
        p = c & 1
        np_ = p ^ 1
        if c + 1 < NCOL:
            if c >= 1:
                drain_scatters(np_)
            gh[np_] = start_gather(np_, c + 1)
        gh[p].wait()
        transpose_rows(p)
        scatter_out(p, c)
    drain_scatters(0)
    drain_scatters(1)


def _gather(idx, w2d):
    mesh = plsc.VectorSubcoreMesh(core_axis_name="c", subcore_axis_name="s")
    k = pl.kernel(
        _gather_body,
        mesh=mesh,
        compiler_params=pltpu.CompilerParams(
            use_tc_tiling_on_sc=False, needs_layout_passes=False),
        out_type=jax.ShapeDtypeStruct((NCOL, 4, 128, 8, 128), jnp.float32),
        scratch_types=[
            pltpu.VMEM((RPT, NCOL), jnp.int32),
            pltpu.VMEM((NCOL * RPT,), jnp.int32),
            pltpu.VMEM((RPT, DIM), jnp.float32),
            pltpu.VMEM((RPT, DIM), jnp.float32),
            pltpu.VMEM((RPT * DIM,), jnp.float32),
            pltpu.VMEM((RPT * DIM,), jnp.float32),
            pltpu.SemaphoreType.DMA,
            pltpu.SemaphoreType.DMA,
            pltpu.SemaphoreType.DMA,
            pltpu.SemaphoreType.DMA,
        ],
    )
    return k(idx, w2d)


def kernel(inputs, weight):
    idx = inputs.astype(jnp.int32)
    tail = lax.slice(weight, (TAIL0, 0), (NUM_EMB, DIM)).reshape(-1)
    w1d = _linearize(weight.T, tail)
    w2d = w1d.reshape(NUM_EMB, DIM)
    out5 = _gather(idx, w2d)
    return jnp.transpose(out5, (2, 4, 0, 1, 3)).reshape(NROW, NCOL, DIM)
